# Initial kernel scaffold; baseline (speedup 1.0000x reference)
import jax
import jax.numpy as jnp
from jax.experimental import pallas as pl


def kernel(x, edge_index, W1, a1_src, a1_dst, b1, W2, a2_src, a2_dst, b2, W3, a3_src, a3_dst, b3):
    raise NotImplementedError("write your pallas kernel here")


# speedup vs baseline: 1.6908x; 1.6908x over previous
import jax
import jax.numpy as jnp
from jax.experimental import pallas as pl
from jax.experimental.pallas import tpu as pltpu


def _pick(n, cap):
    best = 1
    for d in range(1, cap + 1):
        if n % d == 0:
            best = d
    return best


def _dense_stage(x, W, As, Ad, bn):
    N, K = x.shape
    HC = W.shape[1]
    H = As.shape[1]

    def body(x_ref, w_ref, as_ref, ad_ref, h_ref, s_ref, d_ref):
        h = jnp.dot(x_ref[...], w_ref[...], preferred_element_type=jnp.float32)
        h_ref[...] = h
        s_ref[...] = jnp.dot(h, as_ref[...], preferred_element_type=jnp.float32)
        d_ref[...] = jnp.dot(h, ad_ref[...], preferred_element_type=jnp.float32)

    return pl.pallas_call(
        body,
        grid=(N // bn,),
        in_specs=[
            pl.BlockSpec((bn, K), lambda i: (i, 0)),
            pl.BlockSpec((K, HC), lambda i: (0, 0)),
            pl.BlockSpec((HC, H), lambda i: (0, 0)),
            pl.BlockSpec((HC, H), lambda i: (0, 0)),
        ],
        out_specs=[
            pl.BlockSpec((bn, HC), lambda i: (i, 0)),
            pl.BlockSpec((bn, H), lambda i: (i, 0)),
            pl.BlockSpec((bn, H), lambda i: (i, 0)),
        ],
        out_shape=[
            jax.ShapeDtypeStruct((N, HC), jnp.float32),
            jax.ShapeDtypeStruct((N, H), jnp.float32),
            jax.ShapeDtypeStruct((N, H), jnp.float32),
        ],
        compiler_params=pltpu.CompilerParams(
            dimension_semantics=("arbitrary",)),
    )(x, W, As, Ad)


def _edge_stage(asrc, adst, srcb, dstb):
    N, H = asrc.shape
    NB, _, B = srcb.shape

    def body(s_ref, d_ref, as_ref, ad_ref, w_ref, den_ref):
        k = pl.program_id(0)

        @pl.when(k == 0)
        def _():
            den_ref[...] = jnp.zeros_like(den_ref)

        def it(i, carry):
            s = s_ref[0, 0, i]
            d = d_ref[0, 0, i]
            e = as_ref[s, :] + ad_ref[d, :]
            e = jnp.where(e >= 0.0, e, 0.2 * e)
            w = jnp.exp(e)
            w_ref[0, i, :] = w
            den_ref[d, :] = den_ref[d, :] + w
            return carry

        jax.lax.fori_loop(0, B, it, 0)

    return pl.pallas_call(
        body,
        grid=(NB,),
        in_specs=[
            pl.BlockSpec((1, 1, B), lambda k: (k, 0, 0),
                         memory_space=pltpu.SMEM),
            pl.BlockSpec((1, 1, B), lambda k: (k, 0, 0),
                         memory_space=pltpu.SMEM),
            pl.BlockSpec((N, H), lambda k: (0, 0)),
            pl.BlockSpec((N, H), lambda k: (0, 0)),
        ],
        out_specs=[
            pl.BlockSpec((1, B, H), lambda k: (k, 0, 0)),
            pl.BlockSpec((N, H), lambda k: (0, 0)),
        ],
        out_shape=[
            jax.ShapeDtypeStruct((NB, B, H), jnp.float32),
            jax.ShapeDtypeStruct((N, H), jnp.float32),
        ],
        compiler_params=pltpu.CompilerParams(
            dimension_semantics=("arbitrary",)),
    )(srcb, dstb, asrc, adst)


def _agg_stage(hT, wT, srcb, dstb, denT, biasT):
    H, N, C = hT.shape
    NB, _, B = srcb.shape

    def body(s_ref, d_ref, w_ref, h_ref, den_ref, b_ref, o_ref):
        k = pl.program_id(1)

        @pl.when(k == 0)
        def _():
            o_ref[...] = jnp.zeros_like(o_ref)

        def it(i, carry):
            s = s_ref[0, 0, i]
            d = d_ref[0, 0, i]
            wv = w_ref[0, 0, i]
            o_ref[0, d, :] = o_ref[0, d, :] + wv * h_ref[0, s, :]
            return carry

        jax.lax.fori_loop(0, B, it, 0)

        @pl.when(k == NB - 1)
        def _():
            den = den_ref[0]
            o = o_ref[0] / (den + 1e-16) + b_ref[0]
            o_ref[0] = jnp.maximum(o, 0.0)

    return pl.pallas_call(
        body,
        grid=(H, NB),
        in_specs=[
            pl.BlockSpec((1, 1, B), lambda h, k: (k, 0, 0),
                         memory_space=pltpu.SMEM),
            pl.BlockSpec((1, 1, B), lambda h, k: (k, 0, 0),
                         memory_space=pltpu.SMEM),
            pl.BlockSpec((1, 1, B), lambda h, k: (h * NB + k, 0, 0),
                         memory_space=pltpu.SMEM),
            pl.BlockSpec((1, N, C), lambda h, k: (h, 0, 0)),
            pl.BlockSpec((1, N, 1), lambda h, k: (h, 0, 0)),
            pl.BlockSpec((1, 1, C), lambda h, k: (h, 0, 0)),
        ],
        out_specs=pl.BlockSpec((1, N, C), lambda h, k: (h, 0, 0)),
        out_shape=jax.ShapeDtypeStruct((H, N, C), jnp.float32),
        compiler_params=pltpu.CompilerParams(
            dimension_semantics=("arbitrary", "arbitrary")),
    )(srcb, dstb, wT, hT, denT, biasT)


def _expand_attn(a):
    H, C = a.shape
    eye = jnp.eye(H, dtype=a.dtype)
    return (a[:, :, None] * eye[:, None, :]).reshape(H * C, H)


def _gat_layer(x, srcb, dstb, W, a_src, a_dst, b, H, C):
    N = x.shape[0]
    NB, _, B = srcb.shape
    bn = _pick(N, 1024)
    As = _expand_attn(a_src)
    Ad = _expand_attn(a_dst)
    h, al_s, al_d = _dense_stage(x, W, As, Ad, bn)
    w, den = _edge_stage(al_s, al_d, srcb, dstb)
    hT = h.reshape(N, H, C).transpose(1, 0, 2)
    wT = w.transpose(2, 0, 1).reshape(H * NB, 1, B)
    denT = den.T.reshape(H, N, 1)
    biasT = b.reshape(H, 1, C)
    out = _agg_stage(hT, wT, srcb, dstb, denT, biasT)
    return out.transpose(1, 0, 2).reshape(N, H * C)


def kernel(x, edge_index, W1, a1_src, a1_dst, b1, W2, a2_src, a2_dst, b2,
           W3, a3_src, a3_dst, b3):
    N = x.shape[0]
    E = edge_index.shape[1]
    B = _pick(E, 1024)
    NB = E // B
    ei = edge_index.astype(jnp.int32)
    srcb = ei[0].reshape(NB, 1, B)
    dstb = ei[1].reshape(NB, 1, B)

    h = _gat_layer(x, srcb, dstb, W1, a1_src, a1_dst, b1, 8, 128)
    h = _gat_layer(h, srcb, dstb, W2, a2_src, a2_dst, b2, 8, 128)
    h = _gat_layer(h, srcb, dstb, W3, a3_src, a3_dst, b3, 1, 4)
    return h
